# baseline (speedup 1.0000x reference)
import functools

import jax
import jax.numpy as jnp
from jax import lax
from jax.experimental import pallas as pl
from jax.experimental.pallas import tpu as pltpu
from jax.experimental.pallas import tpu_sc as plsc

N = 10000
E = 320000
H = 128
V = 10001
O = 128

NC = 2
NS = 16
NW = NC * NS

NP = 10240
EMB_CH = 80
ENCH0 = 3
ENCH1 = 5
EMAXCH = max(ENCH0, ENCH1)
EROWS0 = ENCH0 * EMB_CH
EROWS1 = ENCH1 * EMB_CH
ESPLIT = NS * EROWS0

ECH = 128
CH0 = 37
CH1 = 120
MAXCH = max(CH0, CH1)
EP = NS * (CH0 + CH1) * ECH

AGG_ROWS = NP
ROWS_S = AGG_ROWS // NS


def _emb_body(kemb, vemb, kidx, vidx, h_out, ki_v, vi_v, kbuf, vbuf, s1, s2):
    cid = lax.axis_index("c")
    sid = lax.axis_index("s")
    wid = sid * NC + cid
    pltpu.sync_copy(kidx.at[wid], ki_v)
    pltpu.sync_copy(vidx.at[wid], vi_v)
    ench = jnp.where(cid == 0, ENCH0, ENCH1)
    base = jnp.where(cid == 0, sid * EROWS0, ESPLIT + sid * EROWS1)

    @pl.loop(0, ench)
    def _chunk(j):
        a = pltpu.async_copy(kemb.at[ki_v.at[j]], kbuf, s1)
        b = pltpu.async_copy(vemb.at[vi_v.at[j]], vbuf, s2)
        a.wait()
        b.wait()

        @pl.loop(0, EMB_CH)
        def _row(r):
            for c in range(H // 16):
                x = kbuf[r, pl.ds(c * 16, 16)] + vbuf[r, pl.ds(c * 16, 16)]
                kbuf[r, pl.ds(c * 16, 16)] = jnp.maximum(x, 0.0)

        row = pl.multiple_of(base + j * EMB_CH, 8)
        pltpu.sync_copy(kbuf, h_out.at[pl.ds(row, EMB_CH)])


def _agg_body(h, src, dst, agg_out, si_v, di_v, gbuf0, agg_sh, s0):
    cid = lax.axis_index("c")
    sid = lax.axis_index("s")
    wid = sid * NC + cid
    pltpu.sync_copy(src.at[wid], si_v)
    pltpu.sync_copy(dst.at[wid], di_v)

    @pl.loop(0, ECH)
    def _zrow(r):
        for c in range(H // 16):
            gbuf0[r, pl.ds(c * 16, 16)] = jnp.zeros((16,), jnp.float32)

    for z in range(ROWS_S // ECH):
        pltpu.sync_copy(gbuf0, agg_sh.at[pl.ds(sid * ROWS_S + z * ECH, ECH)])
    plsc.subcore_barrier()

    nch = jnp.where(cid == 0, CH0, CH1)

    @pl.loop(0, nch)
    def _chunk(j):
        pltpu.async_copy(h.at[si_v.at[j]], gbuf0, s0).wait()
        pltpu.sync_copy(gbuf0, agg_sh.at[di_v.at[j]], add=True)

    plsc.subcore_barrier()

    base = sid * ROWS_S
    for z in range(ROWS_S // ECH):
        pltpu.sync_copy(agg_sh.at[pl.ds(base + z * ECH, ECH)], gbuf0)
        pltpu.sync_copy(gbuf0, agg_out.at[cid, pl.ds(base + z * ECH, ECH)])


def _mlp_body(h_ref, agg_ref, w1_ref, b1_ref, w2_ref, b2_ref, wc_ref, o_ref):
    x = h_ref[...] + agg_ref[0] + agg_ref[1]
    y = jnp.dot(x, w1_ref[...], preferred_element_type=jnp.float32) + b1_ref[...]
    y = jnp.maximum(y, 0.0)
    y = jnp.dot(y, w2_ref[...], preferred_element_type=jnp.float32) + b2_ref[...]
    o_ref[...] = jnp.dot(y, wc_ref[...], preferred_element_type=jnp.float32)


_MESH = plsc.VectorSubcoreMesh(core_axis_name="c", subcore_axis_name="s")

_emb_call = pl.kernel(
    _emb_body,
    out_type=jax.ShapeDtypeStruct((NP, H), jnp.float32),
    mesh=_MESH,
    scratch_types=[
        pltpu.VMEM((EMAXCH, EMB_CH), jnp.int32),
        pltpu.VMEM((EMAXCH, EMB_CH), jnp.int32),
        pltpu.VMEM((EMB_CH, H), jnp.float32),
        pltpu.VMEM((EMB_CH, H), jnp.float32),
        pltpu.SemaphoreType.DMA,
        pltpu.SemaphoreType.DMA,
    ],
)

_agg_call = pl.kernel(
    _agg_body,
    out_type=jax.ShapeDtypeStruct((NC, AGG_ROWS, H), jnp.float32),
    mesh=_MESH,
    scratch_types=[
        pltpu.VMEM((MAXCH, ECH), jnp.int32),
        pltpu.VMEM((MAXCH, ECH), jnp.int32),
        pltpu.VMEM((ECH, H), jnp.float32),
        pltpu.VMEM_SHARED((AGG_ROWS, H), jnp.float32),
        pltpu.SemaphoreType.DMA,
    ],
)

_BLK = 1000
_GRID = N // _BLK


@functools.partial(jax.jit, static_argnames=())
def kernel(edge_index, feats, key_emb, val_emb, W1, b1, W2, b2, Wc):
    def _esplit(x):
        a = x[:ESPLIT].reshape(NS, ENCH0, EMB_CH)
        a = jnp.concatenate(
            [a, jnp.zeros((NS, EMAXCH - ENCH0, EMB_CH), jnp.int32)], axis=1)
        b = x[ESPLIT:].reshape(NS, ENCH1, EMB_CH)
        return jnp.stack([a, b], axis=1).reshape(NW, EMAXCH, EMB_CH)

    kidx = _esplit(jnp.concatenate(
        [feats[:, 0].astype(jnp.int32), jnp.zeros((NP - N,), jnp.int32)]))
    vidx = _esplit(jnp.concatenate(
        [feats[:, 1].astype(jnp.int32), jnp.zeros((NP - N,), jnp.int32)]))
    src = jnp.concatenate(
        [edge_index[0].astype(jnp.int32), jnp.zeros((EP - E,), jnp.int32)])
    dst = jnp.concatenate(
        [edge_index[1].astype(jnp.int32),
         N + jnp.arange(EP - E, dtype=jnp.int32) % (AGG_ROWS - N)])

    def _split(x):
        cap0 = NS * CH0 * ECH
        a = x[:cap0].reshape(NS, CH0, ECH)
        b = x[cap0:].reshape(NS, CH1, ECH)
        a = jnp.concatenate(
            [a, jnp.zeros((NS, MAXCH - CH0, ECH), jnp.int32)], axis=1)
        b = jnp.concatenate(
            [b, jnp.zeros((NS, MAXCH - CH1, ECH), jnp.int32)], axis=1)
        return jnp.stack([a, b], axis=1).reshape(NW, MAXCH, ECH)

    h = _emb_call(key_emb, val_emb, kidx, vidx)
    agg = _agg_call(h, _split(src), _split(dst))

    out = pl.pallas_call(
        _mlp_body,
        grid=(_GRID,),
        in_specs=[
            pl.BlockSpec((_BLK, H), lambda i: (i, 0)),
            pl.BlockSpec((NC, _BLK, H), lambda i: (0, i, 0)),
            pl.BlockSpec((H, H), lambda i: (0, 0)),
            pl.BlockSpec((1, H), lambda i: (0, 0)),
            pl.BlockSpec((H, H), lambda i: (0, 0)),
            pl.BlockSpec((1, H), lambda i: (0, 0)),
            pl.BlockSpec((H, O), lambda i: (0, 0)),
        ],
        out_specs=pl.BlockSpec((_BLK, O), lambda i: (i, 0)),
        out_shape=jax.ShapeDtypeStruct((N, O), jnp.float32),
    )(h, agg, W1, b1.reshape(1, H), W2, b2.reshape(1, H), Wc)
    return out


# speedup vs baseline: 1.2951x; 1.2951x over previous
import functools

import jax
import jax.numpy as jnp
from jax import lax
from jax.experimental import pallas as pl
from jax.experimental.pallas import tpu as pltpu
from jax.experimental.pallas import tpu_sc as plsc

N = 10000
E = 320000
H = 128
V = 10001
O = 128

NC = 2
NS = 16
NW = NC * NS

NP = 10240
EMB_CH = 80
ENCH0 = 5
ENCH1 = 3
EMAXCH = max(ENCH0, ENCH1)
EROWS0 = ENCH0 * EMB_CH
EROWS1 = ENCH1 * EMB_CH
ESPLIT = NS * EROWS0

ECH = 128
CH0 = 82
CH1 = 75
MAXCH = max(CH0, CH1)
EP = NS * (CH0 + CH1) * ECH

AGG_ROWS = NP
ROWS_S = AGG_ROWS // NS


def _emb_body(kemb, vemb, kidx, vidx, h_out, ki_v, vi_v, kbuf, vbuf, s1, s2):
    cid = lax.axis_index("c")
    sid = lax.axis_index("s")
    wid = sid * NC + cid
    pltpu.sync_copy(kidx.at[wid], ki_v)
    pltpu.sync_copy(vidx.at[wid], vi_v)
    ench = jnp.where(cid == 0, ENCH0, ENCH1)
    base = jnp.where(cid == 0, sid * EROWS0, ESPLIT + sid * EROWS1)

    @pl.loop(0, ench)
    def _chunk(j):
        a = pltpu.async_copy(kemb.at[ki_v.at[j]], kbuf, s1)
        b = pltpu.async_copy(vemb.at[vi_v.at[j]], vbuf, s2)
        a.wait()
        b.wait()

        @pl.loop(0, EMB_CH)
        def _row(r):
            for c in range(H // 16):
                x = kbuf[r, pl.ds(c * 16, 16)] + vbuf[r, pl.ds(c * 16, 16)]
                kbuf[r, pl.ds(c * 16, 16)] = jnp.maximum(x, 0.0)

        row = pl.multiple_of(base + j * EMB_CH, 8)
        pltpu.sync_copy(kbuf, h_out.at[pl.ds(row, EMB_CH)])


def _agg_body(h, src, dst, agg_out, si_v, di_v, gbuf0, agg_sh, s0):
    cid = lax.axis_index("c")
    sid = lax.axis_index("s")
    wid = sid * NC + cid
    pltpu.sync_copy(src.at[wid], si_v)
    pltpu.sync_copy(dst.at[wid], di_v)

    @pl.loop(0, ECH)
    def _zrow(r):
        for c in range(H // 16):
            gbuf0[r, pl.ds(c * 16, 16)] = jnp.zeros((16,), jnp.float32)

    for z in range(ROWS_S // ECH):
        pltpu.sync_copy(gbuf0, agg_sh.at[pl.ds(sid * ROWS_S + z * ECH, ECH)])
    plsc.subcore_barrier()

    nch = jnp.where(cid == 0, CH0, CH1)

    @pl.loop(0, nch)
    def _chunk(j):
        pltpu.async_copy(h.at[si_v.at[j]], gbuf0, s0).wait()
        pltpu.sync_copy(gbuf0, agg_sh.at[di_v.at[j]], add=True)

    plsc.subcore_barrier()

    base = sid * ROWS_S
    for z in range(ROWS_S // ECH):
        pltpu.sync_copy(agg_sh.at[pl.ds(base + z * ECH, ECH)], gbuf0)
        pltpu.sync_copy(gbuf0, agg_out.at[cid, pl.ds(base + z * ECH, ECH)])


def _mlp_body(h_ref, agg_ref, w1_ref, b1_ref, w2_ref, b2_ref, wc_ref, o_ref):
    x = h_ref[...] + agg_ref[0] + agg_ref[1]
    y = jnp.dot(x, w1_ref[...], preferred_element_type=jnp.float32) + b1_ref[...]
    y = jnp.maximum(y, 0.0)
    y = jnp.dot(y, w2_ref[...], preferred_element_type=jnp.float32) + b2_ref[...]
    o_ref[...] = jnp.dot(y, wc_ref[...], preferred_element_type=jnp.float32)


_MESH = plsc.VectorSubcoreMesh(core_axis_name="c", subcore_axis_name="s")

_emb_call = pl.kernel(
    _emb_body,
    out_type=jax.ShapeDtypeStruct((NP, H), jnp.float32),
    mesh=_MESH,
    scratch_types=[
        pltpu.VMEM((EMAXCH, EMB_CH), jnp.int32),
        pltpu.VMEM((EMAXCH, EMB_CH), jnp.int32),
        pltpu.VMEM((EMB_CH, H), jnp.float32),
        pltpu.VMEM((EMB_CH, H), jnp.float32),
        pltpu.SemaphoreType.DMA,
        pltpu.SemaphoreType.DMA,
    ],
)

_agg_call = pl.kernel(
    _agg_body,
    out_type=jax.ShapeDtypeStruct((NC, AGG_ROWS, H), jnp.float32),
    mesh=_MESH,
    scratch_types=[
        pltpu.VMEM((MAXCH, ECH), jnp.int32),
        pltpu.VMEM((MAXCH, ECH), jnp.int32),
        pltpu.VMEM((ECH, H), jnp.float32),
        pltpu.VMEM_SHARED((AGG_ROWS, H), jnp.float32),
        pltpu.SemaphoreType.DMA,
    ],
)

_BLK = 1000
_GRID = N // _BLK


@functools.partial(jax.jit, static_argnames=())
def kernel(edge_index, feats, key_emb, val_emb, W1, b1, W2, b2, Wc):
    def _pad_chunks(x, maxch):
        n = x.shape[1]
        if n == maxch:
            return x
        return jnp.concatenate(
            [x, jnp.zeros((NS, maxch - n, x.shape[2]), jnp.int32)], axis=1)

    def _esplit(x):
        a = _pad_chunks(x[:ESPLIT].reshape(NS, ENCH0, EMB_CH), EMAXCH)
        b = _pad_chunks(x[ESPLIT:].reshape(NS, ENCH1, EMB_CH), EMAXCH)
        return jnp.stack([a, b], axis=1).reshape(NW, EMAXCH, EMB_CH)

    kidx = _esplit(jnp.concatenate(
        [feats[:, 0].astype(jnp.int32), jnp.zeros((NP - N,), jnp.int32)]))
    vidx = _esplit(jnp.concatenate(
        [feats[:, 1].astype(jnp.int32), jnp.zeros((NP - N,), jnp.int32)]))
    src = jnp.concatenate(
        [edge_index[0].astype(jnp.int32), jnp.zeros((EP - E,), jnp.int32)])
    dst = jnp.concatenate(
        [edge_index[1].astype(jnp.int32),
         N + jnp.arange(EP - E, dtype=jnp.int32) % (AGG_ROWS - N)])

    def _split(x):
        cap0 = NS * CH0 * ECH
        a = _pad_chunks(x[:cap0].reshape(NS, CH0, ECH), MAXCH)
        b = _pad_chunks(x[cap0:].reshape(NS, CH1, ECH), MAXCH)
        return jnp.stack([a, b], axis=1).reshape(NW, MAXCH, ECH)

    h = _emb_call(key_emb, val_emb, kidx, vidx)
    agg = _agg_call(h, _split(src), _split(dst))

    out = pl.pallas_call(
        _mlp_body,
        grid=(_GRID,),
        in_specs=[
            pl.BlockSpec((_BLK, H), lambda i: (i, 0)),
            pl.BlockSpec((NC, _BLK, H), lambda i: (0, i, 0)),
            pl.BlockSpec((H, H), lambda i: (0, 0)),
            pl.BlockSpec((1, H), lambda i: (0, 0)),
            pl.BlockSpec((H, H), lambda i: (0, 0)),
            pl.BlockSpec((1, H), lambda i: (0, 0)),
            pl.BlockSpec((H, O), lambda i: (0, 0)),
        ],
        out_specs=pl.BlockSpec((_BLK, O), lambda i: (i, 0)),
        out_shape=jax.ShapeDtypeStruct((N, O), jnp.float32),
    )(h, agg, W1, b1.reshape(1, H), W2, b2.reshape(1, H), Wc)
    return out


# speedup vs baseline: 1.3616x; 1.0514x over previous
import functools

import jax
import jax.numpy as jnp
from jax import lax
from jax.experimental import pallas as pl
from jax.experimental.pallas import tpu as pltpu
from jax.experimental.pallas import tpu_sc as plsc

N = 10000
E = 320000
H = 128
V = 10001
O = 128

NC = 2
NS = 16
NW = NC * NS

NP = 10240
EMB_CH = 80
ENCH0 = 5
ENCH1 = 3
EMAXCH = max(ENCH0, ENCH1)
EROWS0 = ENCH0 * EMB_CH
EROWS1 = ENCH1 * EMB_CH
ESPLIT = NS * EROWS0

ECH = 128
CH0 = 90
CH1 = 67
MAXCH = max(CH0, CH1)
EP = NS * (CH0 + CH1) * ECH

AGG_ROWS = NP
ROWS_S = AGG_ROWS // NS


def _emb_body(kemb, vemb, kidx, vidx, h_out, ki_v, vi_v, kbuf, vbuf, s1, s2):
    cid = lax.axis_index("c")
    sid = lax.axis_index("s")
    wid = sid * NC + cid
    pltpu.sync_copy(kidx.at[wid], ki_v)
    pltpu.sync_copy(vidx.at[wid], vi_v)
    ench = jnp.where(cid == 0, ENCH0, ENCH1)
    base = jnp.where(cid == 0, sid * EROWS0, ESPLIT + sid * EROWS1)

    @pl.loop(0, ench)
    def _chunk(j):
        a = pltpu.async_copy(kemb.at[ki_v.at[j]], kbuf, s1)
        b = pltpu.async_copy(vemb.at[vi_v.at[j]], vbuf, s2)
        a.wait()
        b.wait()

        @pl.loop(0, EMB_CH)
        def _row(r):
            for c in range(H // 16):
                x = kbuf[r, pl.ds(c * 16, 16)] + vbuf[r, pl.ds(c * 16, 16)]
                kbuf[r, pl.ds(c * 16, 16)] = jnp.maximum(x, 0.0)

        row = pl.multiple_of(base + j * EMB_CH, 8)
        pltpu.sync_copy(kbuf, h_out.at[pl.ds(row, EMB_CH)])


def _agg_body(h, src, dst, agg_out, si_v, di_v, gbuf0, agg_sh, s0):
    cid = lax.axis_index("c")
    sid = lax.axis_index("s")
    wid = sid * NC + cid
    pltpu.sync_copy(src.at[wid], si_v)
    pltpu.sync_copy(dst.at[wid], di_v)

    @pl.loop(0, ECH)
    def _zrow(r):
        for c in range(H // 16):
            gbuf0[r, pl.ds(c * 16, 16)] = jnp.zeros((16,), jnp.float32)

    for z in range(ROWS_S // ECH):
        pltpu.sync_copy(gbuf0, agg_sh.at[pl.ds(sid * ROWS_S + z * ECH, ECH)])
    plsc.subcore_barrier()

    nch = jnp.where(cid == 0, CH0, CH1)

    @pl.loop(0, nch)
    def _chunk(j):
        pltpu.async_copy(h.at[si_v.at[j]], gbuf0, s0).wait()
        pltpu.sync_copy(gbuf0, agg_sh.at[di_v.at[j]], add=True)

    plsc.subcore_barrier()

    base = sid * ROWS_S
    for z in range(ROWS_S // ECH):
        pltpu.sync_copy(agg_sh.at[pl.ds(base + z * ECH, ECH)], gbuf0)
        pltpu.sync_copy(gbuf0, agg_out.at[cid, pl.ds(base + z * ECH, ECH)])


def _mlp_body(h_ref, agg_ref, w1_ref, b1_ref, w2_ref, b2_ref, wc_ref, o_ref):
    x = h_ref[...] + agg_ref[0] + agg_ref[1]
    y = jnp.dot(x, w1_ref[...], preferred_element_type=jnp.float32) + b1_ref[...]
    y = jnp.maximum(y, 0.0)
    y = jnp.dot(y, w2_ref[...], preferred_element_type=jnp.float32) + b2_ref[...]
    o_ref[...] = jnp.dot(y, wc_ref[...], preferred_element_type=jnp.float32)


_MESH = plsc.VectorSubcoreMesh(core_axis_name="c", subcore_axis_name="s")

_emb_call = pl.kernel(
    _emb_body,
    out_type=jax.ShapeDtypeStruct((NP, H), jnp.float32),
    mesh=_MESH,
    scratch_types=[
        pltpu.VMEM((EMAXCH, EMB_CH), jnp.int32),
        pltpu.VMEM((EMAXCH, EMB_CH), jnp.int32),
        pltpu.VMEM((EMB_CH, H), jnp.float32),
        pltpu.VMEM((EMB_CH, H), jnp.float32),
        pltpu.SemaphoreType.DMA,
        pltpu.SemaphoreType.DMA,
    ],
)

_agg_call = pl.kernel(
    _agg_body,
    out_type=jax.ShapeDtypeStruct((NC, AGG_ROWS, H), jnp.float32),
    mesh=_MESH,
    scratch_types=[
        pltpu.VMEM((MAXCH, ECH), jnp.int32),
        pltpu.VMEM((MAXCH, ECH), jnp.int32),
        pltpu.VMEM((ECH, H), jnp.float32),
        pltpu.VMEM_SHARED((AGG_ROWS, H), jnp.float32),
        pltpu.SemaphoreType.DMA,
    ],
)

_BLK = 1000
_GRID = N // _BLK


@functools.partial(jax.jit, static_argnames=())
def kernel(edge_index, feats, key_emb, val_emb, W1, b1, W2, b2, Wc):
    def _pad_chunks(x, maxch):
        n = x.shape[1]
        if n == maxch:
            return x
        return jnp.concatenate(
            [x, jnp.zeros((NS, maxch - n, x.shape[2]), jnp.int32)], axis=1)

    def _esplit(x):
        a = _pad_chunks(x[:ESPLIT].reshape(NS, ENCH0, EMB_CH), EMAXCH)
        b = _pad_chunks(x[ESPLIT:].reshape(NS, ENCH1, EMB_CH), EMAXCH)
        return jnp.stack([a, b], axis=1).reshape(NW, EMAXCH, EMB_CH)

    kidx = _esplit(jnp.concatenate(
        [feats[:, 0].astype(jnp.int32), jnp.zeros((NP - N,), jnp.int32)]))
    vidx = _esplit(jnp.concatenate(
        [feats[:, 1].astype(jnp.int32), jnp.zeros((NP - N,), jnp.int32)]))
    src = jnp.concatenate(
        [edge_index[0].astype(jnp.int32), jnp.zeros((EP - E,), jnp.int32)])
    dst = jnp.concatenate(
        [edge_index[1].astype(jnp.int32),
         N + jnp.arange(EP - E, dtype=jnp.int32) % (AGG_ROWS - N)])

    def _split(x):
        cap0 = NS * CH0 * ECH
        a = _pad_chunks(x[:cap0].reshape(NS, CH0, ECH), MAXCH)
        b = _pad_chunks(x[cap0:].reshape(NS, CH1, ECH), MAXCH)
        return jnp.stack([a, b], axis=1).reshape(NW, MAXCH, ECH)

    h = _emb_call(key_emb, val_emb, kidx, vidx)
    agg = _agg_call(h, _split(src), _split(dst))

    out = pl.pallas_call(
        _mlp_body,
        grid=(_GRID,),
        in_specs=[
            pl.BlockSpec((_BLK, H), lambda i: (i, 0)),
            pl.BlockSpec((NC, _BLK, H), lambda i: (0, i, 0)),
            pl.BlockSpec((H, H), lambda i: (0, 0)),
            pl.BlockSpec((1, H), lambda i: (0, 0)),
            pl.BlockSpec((H, H), lambda i: (0, 0)),
            pl.BlockSpec((1, H), lambda i: (0, 0)),
            pl.BlockSpec((H, O), lambda i: (0, 0)),
        ],
        out_specs=pl.BlockSpec((_BLK, O), lambda i: (i, 0)),
        out_shape=jax.ShapeDtypeStruct((N, O), jnp.float32),
    )(h, agg, W1, b1.reshape(1, H), W2, b2.reshape(1, H), Wc)
    return out
